# baseline (device time: 16168 ns/iter reference)
import jax
import jax.numpy as jnp
from jax import lax
from jax.experimental import pallas as pl
from jax.experimental.pallas import tpu as pltpu

N_DEV = 4
B, Sq, Skv, Hq, Dh = 2, 128, 512, 4, 64
SKV_LOC = Skv // N_DEV
PH1_ROWS = {0: (0, 128), 1: (0, 128), 2: (64, 64), 3: (0, 64)}
WAIT_ORDER = (1, 3, 2)


def kernel(x, Wq, K_ext, V_ext, Wo):
    x2 = x.reshape(B * Sq, 512)
    k2 = K_ext.reshape(B, SKV_LOC, Hq * Dh)
    v2 = V_ext.reshape(B, SKV_LOC, Hq * Dh)

    def body(x_ref, wq_ref, k_ref, v_ref, wo_ref, out_ref,
             kvbf, kvh, ctx_gather,
             ph1_send, ph1_recv, ph2_send, ph2_recv):
        my = lax.axis_index("i")
        b = my // 2
        hb = 2 * (my % 2)

        barrier_sem = pltpu.get_barrier_semaphore()
        for k in range(1, N_DEV):
            pl.semaphore_signal(
                barrier_sem, inc=1,
                device_id=((my + k) % N_DEV,),
                device_id_type=pl.DeviceIdType.MESH,
            )

        for h in range(Hq):
            kvbf[0, :, h] = k_ref[:, :, h * Dh:(h + 1) * Dh].astype(
                jnp.bfloat16)
            kvbf[1, :, h] = v_ref[:, :, h * Dh:(h + 1) * Dh].astype(
                jnp.bfloat16)

        kvh[:, :, 256:320, :] = jnp.zeros((2, 2, 64, Dh), jnp.bfloat16)
        kvh[:, :, 448:512, :] = jnp.zeros((2, 2, 64, Dh), jnp.bfloat16)

        pl.semaphore_wait(barrier_sem, N_DEV - 1)

        def ph1_send_descs(row0, nrows, slot):
            descs = []
            for k in range(1, N_DEV):
                t = (my + k) % N_DEV
                b_t = t // 2
                hb_t = 2 * (t % 2)
                descs.append(pltpu.make_async_remote_copy(
                    src_ref=kvbf.at[:, b_t, pl.ds(hb_t, 2),
                                    pl.ds(row0, nrows), :],
                    dst_ref=kvh.at[:, :, pl.ds(slot * SKV_LOC + row0,
                                               nrows), :],
                    send_sem=ph1_send.at[k - 1],
                    recv_sem=ph1_recv.at[slot],
                    device_id=(t,),
                    device_id_type=pl.DeviceIdType.MESH,
                ))
            return descs

        @pl.when(my < 2)
        def _():
            for d in ph1_send_descs(0, SKV_LOC, my):
                d.start()

        @pl.when(my == 2)
        def _():
            for d in ph1_send_descs(64, 64, 2):
                d.start()

        @pl.when(my == 3)
        def _():
            for d in ph1_send_descs(0, 64, 3):
                d.start()

        kvh[:, :, pl.ds(my * SKV_LOC, SKV_LOC), :] = kvbf[:, b, pl.ds(hb, 2)]

        q_mine = (lax.dot_general(
            x_ref[pl.ds(b * Sq, Sq), :],
            wq_ref[:, pl.ds(hb * Dh, 2 * Dh)],
            (((1,), (0,)), ((), ())),
            preferred_element_type=jnp.float32,
        ) * 0.125).astype(jnp.bfloat16)

        qb = lax.broadcasted_iota(jnp.int32, (Sq, Skv), 0) // 64
        kb = lax.broadcasted_iota(jnp.int32, (Sq, Skv), 1) // 64
        mask = (qb == kb) | (kb == 0) | ((qb + kb) % 3 == 0)

        for o in range(N_DEV):
            @pl.when(o != my)
            def _(o=o):
                row0, nrows = PH1_ROWS[o]
                r = pltpu.make_async_remote_copy(
                    src_ref=kvbf.at[:, 0, pl.ds(0, 2),
                                    pl.ds(row0, nrows), :],
                    dst_ref=kvh.at[:, :, pl.ds(o * SKV_LOC + row0,
                                               nrows), :],
                    send_sem=ph1_send.at[0],
                    recv_sem=ph1_recv.at[o],
                    device_id=(o,),
                    device_id_type=pl.DeviceIdType.MESH,
                )
                r.wait_recv()

        ph2_descs = []
        for j in range(2):
            q_j = q_mine[:, j * Dh:(j + 1) * Dh]
            scores = lax.dot_general(
                q_j, kvh[0, j], (((1,), (1,)), ((), ())),
                preferred_element_type=jnp.float32,
            )
            scores = jnp.where(mask, scores, -1e9)
            m = jnp.max(scores, axis=1, keepdims=True)
            w = jnp.exp(scores - m)
            denom = jnp.sum(w, axis=1, keepdims=True)
            ctx_j = lax.dot_general(
                w.astype(jnp.bfloat16), kvh[1, j],
                (((1,), (0,)), ((), ())),
                preferred_element_type=jnp.float32,
            ) * (1.0 / denom)
            ctx_gather[my, j] = ctx_j.astype(jnp.bfloat16)
            for k in range(1, N_DEV):
                t = (my + k) % N_DEV
                d = pltpu.make_async_remote_copy(
                    src_ref=ctx_gather.at[my, j],
                    dst_ref=ctx_gather.at[my, j],
                    send_sem=ph2_send.at[(k - 1) * 2 + j],
                    recv_sem=ph2_recv.at[my, j],
                    device_id=(t,),
                    device_id_type=pl.DeviceIdType.MESH,
                )
                d.start()
                ph2_descs.append(d)

        out_ref[pl.ds((1 - b) * Sq, Sq), :] = jnp.zeros((Sq, 512), jnp.float32)
        for j in range(2):
            part = lax.dot_general(
                ctx_gather[my, j].astype(jnp.float32),
                wo_ref[pl.ds((hb + j) * Dh, Dh), :],
                (((1,), (0,)), ((), ())),
                preferred_element_type=jnp.float32,
            )
            if j == 0:
                out_ref[pl.ds(b * Sq, Sq), :] = part
            else:
                out_ref[pl.ds(b * Sq, Sq), :] += part

        for k in WAIT_ORDER:
            t = (my + k) % N_DEV
            b_t = t // 2
            hb_t = 2 * (t % 2)
            for j in range(2):
                r = pltpu.make_async_remote_copy(
                    src_ref=ctx_gather.at[my, j],
                    dst_ref=ctx_gather.at[t, j],
                    send_sem=ph2_send.at[(k - 1) * 2 + j],
                    recv_sem=ph2_recv.at[t, j],
                    device_id=(t,),
                    device_id_type=pl.DeviceIdType.MESH,
                )
                r.wait_recv()
                out_ref[pl.ds(b_t * Sq, Sq), :] += lax.dot_general(
                    ctx_gather[t, j].astype(jnp.float32),
                    wo_ref[pl.ds((hb_t + j) * Dh, Dh), :],
                    (((1,), (0,)), ((), ())),
                    preferred_element_type=jnp.float32,
                )

        @pl.when(my < 2)
        def _():
            for d in ph1_send_descs(0, SKV_LOC, my):
                d.wait_send()

        @pl.when(my == 2)
        def _():
            for d in ph1_send_descs(64, 64, 2):
                d.wait_send()

        @pl.when(my == 3)
        def _():
            for d in ph1_send_descs(0, 64, 3):
                d.wait_send()

        for d in ph2_descs:
            d.wait_send()

    out2 = pl.pallas_call(
        body,
        out_shape=jax.ShapeDtypeStruct((B * Sq, 512), jnp.float32),
        in_specs=[pl.BlockSpec(memory_space=pltpu.VMEM)] * 5,
        out_specs=pl.BlockSpec(memory_space=pltpu.VMEM),
        scratch_shapes=[
            pltpu.VMEM((2, B, Hq, SKV_LOC, Dh), jnp.bfloat16),
            pltpu.VMEM((2, 2, Skv, Dh), jnp.bfloat16),
            pltpu.VMEM((N_DEV, 2, Sq, Dh), jnp.bfloat16),
            pltpu.SemaphoreType.DMA((N_DEV - 1,)),
            pltpu.SemaphoreType.DMA((N_DEV,)),
            pltpu.SemaphoreType.DMA((2 * (N_DEV - 1),)),
            pltpu.SemaphoreType.DMA((N_DEV, 2)),
        ],
        compiler_params=pltpu.CompilerParams(collective_id=0),
    )(x2, Wq, k2, v2, Wo)

    return out2.reshape(B, Sq, 512)


# device time: 14995 ns/iter; 1.0782x vs baseline; 1.0782x over previous
import jax
import jax.numpy as jnp
from jax import lax
from jax.experimental import pallas as pl
from jax.experimental.pallas import tpu as pltpu

N_DEV = 4
B, Sq, Skv, Hq, Dh = 2, 128, 512, 4, 64
SKV_LOC = Skv // N_DEV
PH1_ROWS = {0: (0, 128), 1: (0, 128), 2: (64, 64), 3: (0, 64)}
WAIT_ORDER = (1, 3, 2)


def kernel(x, Wq, K_ext, V_ext, Wo):
    x2 = x.reshape(B * Sq, 512)
    k2 = K_ext.reshape(B, SKV_LOC, Hq * Dh)
    v2 = V_ext.reshape(B, SKV_LOC, Hq * Dh)

    def body(x_ref, wq_ref, k_ref, v_ref, wo_ref, out_ref,
             kvbf, kvh, ctx_gather,
             ph1_send, ph1_recv, ph2_send, ph2_recv):
        my = lax.axis_index("i")
        b = my // 2
        hb = 2 * (my % 2)

        barrier_sem = pltpu.get_barrier_semaphore()
        for k in range(1, N_DEV):
            pl.semaphore_signal(
                barrier_sem, inc=1,
                device_id=((my + k) % N_DEV,),
                device_id_type=pl.DeviceIdType.MESH,
            )

        kvbf[0] = k_ref[...].astype(jnp.bfloat16)
        kvbf[1] = v_ref[...].astype(jnp.bfloat16)

        kvh[2, :, 0:64, :] = jnp.zeros((2, 64, 2 * Dh), jnp.bfloat16)
        kvh[3, :, 64:128, :] = jnp.zeros((2, 64, 2 * Dh), jnp.bfloat16)

        pl.semaphore_wait(barrier_sem, N_DEV - 1)

        def ph1_send_descs(row0, nrows, slot):
            descs = []
            for k in range(1, N_DEV):
                t = (my + k) % N_DEV
                b_t = t // 2
                hb_t = 2 * (t % 2)
                dst = (kvh.at[slot] if nrows == SKV_LOC
                       else kvh.at[slot, :, pl.ds(row0, nrows), :])
                descs.append(pltpu.make_async_remote_copy(
                    src_ref=kvbf.at[:, b_t, pl.ds(row0, nrows),
                                    pl.ds(hb_t * Dh, 2 * Dh)],
                    dst_ref=dst,
                    send_sem=ph1_send.at[k - 1],
                    recv_sem=ph1_recv.at[slot],
                    device_id=(t,),
                    device_id_type=pl.DeviceIdType.MESH,
                ))
            return descs

        @pl.when(my < 2)
        def _():
            for d in ph1_send_descs(0, SKV_LOC, my):
                d.start()

        @pl.when(my == 2)
        def _():
            for d in ph1_send_descs(64, 64, 2):
                d.start()

        @pl.when(my == 3)
        def _():
            for d in ph1_send_descs(0, 64, 3):
                d.start()

        kvh[my] = kvbf[:, b, :, pl.ds(hb * Dh, 2 * Dh)]

        q_mine = lax.dot_general(
            x_ref[pl.ds(b * Sq, Sq), :],
            wq_ref[:, pl.ds(hb * Dh, 2 * Dh)],
            (((1,), (0,)), ((), ())),
            preferred_element_type=jnp.float32,
        ).astype(jnp.bfloat16)

        qb = lax.broadcasted_iota(jnp.int32, (Sq, Skv), 0) // 64
        kb = lax.broadcasted_iota(jnp.int32, (Sq, Skv), 1) // 64
        mask = (qb == kb) | (kb == 0) | ((qb + kb) % 3 == 0)

        for o in range(N_DEV):
            @pl.when(o != my)
            def _(o=o):
                row0, nrows = PH1_ROWS[o]
                dst = (kvh.at[o] if nrows == SKV_LOC
                       else kvh.at[o, :, pl.ds(row0, nrows), :])
                r = pltpu.make_async_remote_copy(
                    src_ref=kvbf.at[:, 0, pl.ds(row0, nrows),
                                    pl.ds(0, 2 * Dh)],
                    dst_ref=dst,
                    send_sem=ph1_send.at[0],
                    recv_sem=ph1_recv.at[o],
                    device_id=(o,),
                    device_id_type=pl.DeviceIdType.MESH,
                )
                r.wait_recv()

        ph2_descs = []
        for j in range(2):
            q_j = q_mine[:, j * Dh:(j + 1) * Dh]
            scores = jnp.concatenate(
                [
                    lax.dot_general(
                        q_j, kvh[o, 0, :, j * Dh:(j + 1) * Dh],
                        (((1,), (1,)), ((), ())),
                        preferred_element_type=jnp.float32,
                    )
                    for o in range(N_DEV)
                ],
                axis=1,
            )
            scores = jnp.where(mask, scores * 0.125, -1e9)
            m = jnp.max(scores, axis=1, keepdims=True)
            w = jnp.exp(scores - m)
            denom = jnp.sum(w, axis=1, keepdims=True)
            w = w.astype(jnp.bfloat16)
            ctx_j = sum(
                lax.dot_general(
                    w[:, o * SKV_LOC:(o + 1) * SKV_LOC],
                    kvh[o, 1, :, j * Dh:(j + 1) * Dh],
                    (((1,), (0,)), ((), ())),
                    preferred_element_type=jnp.float32,
                )
                for o in range(N_DEV)
            ) * (1.0 / denom)
            ctx_gather[my, j] = ctx_j.astype(jnp.bfloat16)
            for k in range(1, N_DEV):
                t = (my + k) % N_DEV
                d = pltpu.make_async_remote_copy(
                    src_ref=ctx_gather.at[my, j],
                    dst_ref=ctx_gather.at[my, j],
                    send_sem=ph2_send.at[(k - 1) * 2 + j],
                    recv_sem=ph2_recv.at[my, j],
                    device_id=(t,),
                    device_id_type=pl.DeviceIdType.MESH,
                )
                d.start()
                ph2_descs.append(d)

        out_ref[pl.ds((1 - b) * Sq, Sq), :] = jnp.zeros((Sq, 512), jnp.float32)
        for j in range(2):
            part = lax.dot_general(
                ctx_gather[my, j].astype(jnp.float32),
                wo_ref[pl.ds((hb + j) * Dh, Dh), :],
                (((1,), (0,)), ((), ())),
                preferred_element_type=jnp.float32,
            )
            if j == 0:
                out_ref[pl.ds(b * Sq, Sq), :] = part
            else:
                out_ref[pl.ds(b * Sq, Sq), :] += part

        for k in WAIT_ORDER:
            t = (my + k) % N_DEV
            b_t = t // 2
            hb_t = 2 * (t % 2)
            for j in range(2):
                r = pltpu.make_async_remote_copy(
                    src_ref=ctx_gather.at[my, j],
                    dst_ref=ctx_gather.at[t, j],
                    send_sem=ph2_send.at[(k - 1) * 2 + j],
                    recv_sem=ph2_recv.at[t, j],
                    device_id=(t,),
                    device_id_type=pl.DeviceIdType.MESH,
                )
                r.wait_recv()
                out_ref[pl.ds(b_t * Sq, Sq), :] += lax.dot_general(
                    ctx_gather[t, j].astype(jnp.float32),
                    wo_ref[pl.ds((hb_t + j) * Dh, Dh), :],
                    (((1,), (0,)), ((), ())),
                    preferred_element_type=jnp.float32,
                )

        @pl.when(my < 2)
        def _():
            for d in ph1_send_descs(0, SKV_LOC, my):
                d.wait_send()

        @pl.when(my == 2)
        def _():
            for d in ph1_send_descs(64, 64, 2):
                d.wait_send()

        @pl.when(my == 3)
        def _():
            for d in ph1_send_descs(0, 64, 3):
                d.wait_send()

        for d in ph2_descs:
            d.wait_send()

    out2 = pl.pallas_call(
        body,
        out_shape=jax.ShapeDtypeStruct((B * Sq, 512), jnp.float32),
        in_specs=[pl.BlockSpec(memory_space=pltpu.VMEM)] * 5,
        out_specs=pl.BlockSpec(memory_space=pltpu.VMEM),
        scratch_shapes=[
            pltpu.VMEM((2, B, SKV_LOC, Hq * Dh), jnp.bfloat16),
            pltpu.VMEM((N_DEV, 2, SKV_LOC, 2 * Dh), jnp.bfloat16),
            pltpu.VMEM((N_DEV, 2, Sq, Dh), jnp.bfloat16),
            pltpu.SemaphoreType.DMA((N_DEV - 1,)),
            pltpu.SemaphoreType.DMA((N_DEV,)),
            pltpu.SemaphoreType.DMA((2 * (N_DEV - 1),)),
            pltpu.SemaphoreType.DMA((N_DEV, 2)),
        ],
        compiler_params=pltpu.CompilerParams(collective_id=0),
    )(x2, Wq, k2, v2, Wo)

    return out2.reshape(B, Sq, 512)
